# baseline (speedup 1.0000x reference)
import jax
import jax.numpy as jnp
from jax.experimental import pallas as pl
from jax.experimental.pallas import tpu as pltpu


def _fused_kernel(x_ref, w_ref, a_ref, o_ref, t_ref):
    @pl.when(pl.program_id(0) == 0)
    def _compute_t():
        n = x_ref.shape[0]
        w = w_ref[...].astype(jnp.bfloat16)
        chunk = n // 8
        for c in range(8):
            rows = pl.ds(c * chunk, chunk)
            t = jax.lax.dot_general(
                x_ref[rows, :].astype(jnp.bfloat16), w,
                (((1,), (0,)), ((), ())),
                preferred_element_type=jnp.float32)
            t_ref[rows, :] = t.astype(jnp.bfloat16)

    a = a_ref[...].astype(jnp.bfloat16)
    acc = jax.lax.dot_general(
        a, t_ref[...], (((1,), (0,)), ((), ())),
        preferred_element_type=jnp.float32)
    o_ref[...] = jnp.maximum(acc, 0.0)


def kernel(x_source, neighborhood, W1, W2, W3):
    n, d_in = x_source.shape
    d_out = W1.shape[1]
    bi = min(256, n)

    out = pl.pallas_call(
        _fused_kernel,
        grid=(n // bi,),
        in_specs=[pl.BlockSpec((n, d_in), lambda i: (0, 0)),
                  pl.BlockSpec((d_in, d_out), lambda i: (0, 0)),
                  pl.BlockSpec((bi, n), lambda i: (i, 0))],
        out_specs=pl.BlockSpec((bi, d_out), lambda i: (i, 0)),
        out_shape=jax.ShapeDtypeStruct((n, d_out), jnp.float32),
        scratch_shapes=[pltpu.VMEM((n, d_out), jnp.bfloat16)],
    )(x_source, W1, neighborhood)
    return out
